# baseline (speedup 1.0000x reference)
import functools

import jax
import jax.numpy as jnp
from jax import lax
from jax.experimental import pallas as pl
from jax.experimental.pallas import tpu as pltpu
from jax.experimental.pallas import tpu_sc as plsc

D_MODEL = 128
SEQ = 50
NUM_WORKERS = 32
CHUNK = 200
NBUF = 4
P_SPLIT = 4
TC_BK = 8


def _positional_encoding(seq, d_model):
    pos = jnp.arange(seq, dtype=jnp.float32)[:, None]
    i = jnp.arange(0, d_model, 2, dtype=jnp.float32)
    div = jnp.exp(-i * (jnp.log(10000.0) / d_model))
    ang = pos * div
    pe = jnp.zeros((seq, d_model), dtype=jnp.float32)
    pe = pe.at[:, 0::2].set(jnp.sin(ang))
    pe = pe.at[:, 1::2].set(jnp.cos(ang))
    return pe


def _make_sc_gather(n_rows, n_chunks):
    mesh = plsc.VectorSubcoreMesh(core_axis_name="c", subcore_axis_name="s")
    rows_per_w = n_rows // NUM_WORKERS

    @functools.partial(
        pl.kernel,
        mesh=mesh,
        out_type=jax.ShapeDtypeStruct((n_rows, D_MODEL), jnp.float32),
        scratch_types=[pltpu.VMEM((CHUNK,), jnp.int32) for _ in range(NBUF)]
        + [pltpu.VMEM((CHUNK, D_MODEL), jnp.float32) for _ in range(NBUF)]
        + [pltpu.SemaphoreType.DMA for _ in range(3 * NBUF)],
    )
    def sc_gather(x_hbm, tab_hbm, out_hbm, *bufs_sems):
        ibufs = bufs_sems[:NBUF]
        bufs = bufs_sems[NBUF:2 * NBUF]
        isem = bufs_sems[2 * NBUF:3 * NBUF]
        gsem = bufs_sems[3 * NBUF:4 * NBUF]
        ssem = bufs_sems[4 * NBUF:]
        cid = lax.axis_index("c")
        sid = lax.axis_index("s")
        w = sid * 2 + cid
        base = w * rows_per_w

        def start_idx(b, c):
            pltpu.async_copy(x_hbm.at[w, c], ibufs[b], isem[b])

        def wait_idx(b):
            pltpu.make_async_copy(x_hbm.at[w, 0], ibufs[b], isem[b]).wait()

        def start_gather(b):
            pltpu.async_copy(tab_hbm.at[ibufs[b]], bufs[b], gsem[b])

        def wait_gather(b):
            pltpu.make_async_copy(tab_hbm.at[ibufs[b]], bufs[b],
                                  gsem[b]).wait()

        def start_scatter(b, c):
            pltpu.async_copy(bufs[b],
                             out_hbm.at[pl.ds(base + c * CHUNK, CHUNK)],
                             ssem[b])

        def wait_scatter(b):
            pltpu.make_async_copy(bufs[b], out_hbm.at[pl.ds(base, CHUNK)],
                                  ssem[b]).wait()

        for b in range(NBUF):
            start_idx(b, b)
        for b in range(NBUF - 1):
            wait_idx(b)
            start_gather(b)

        def outer_body(g, carry):
            for b in range(NBUF):
                c = g * NBUF + b
                nb = (b + NBUF - 1) % NBUF
                @pl.when(c >= 1)
                def _():
                    wait_scatter(nb)

                @pl.when(c + NBUF - 1 < n_chunks)
                def _():
                    wait_idx(nb)
                    start_gather(nb)

                wait_gather(b)

                @pl.when(c + NBUF < n_chunks)
                def _():
                    start_idx(b, c + NBUF)

                start_scatter(b, c)
            return carry

        lax.fori_loop(0, n_chunks // NBUF, outer_body, 0)
        wait_scatter((n_chunks - 1) % NBUF)

    return sc_gather


def _tc_add_body(g_ref, pe_ref, o_ref):
    for k in range(TC_BK):
        o_ref[k] = g_ref[pl.ds(k * SEQ, SEQ), :] + pe_ref[...]


def _tc_add_first_body(g_ref, pe_ref, o_ref):
    _tc_add_body(g_ref, pe_ref, o_ref)


def _tc_add_alias_body(g_ref, pe_ref, _prev_ref, o_ref):
    _tc_add_body(g_ref, pe_ref, o_ref)


def _tc_add(batch, part_rows, p_off_blocks, g_part, pe, prev):
    n_blocks = part_rows // (TC_BK * SEQ)
    out_shape = jax.ShapeDtypeStruct((batch, SEQ, D_MODEL), jnp.float32)
    g_spec = pl.BlockSpec((TC_BK * SEQ, D_MODEL), lambda i: (i, 0))
    pe_spec = pl.BlockSpec((SEQ, D_MODEL), lambda i: (0, 0))
    o_spec = pl.BlockSpec((TC_BK, SEQ, D_MODEL),
                          lambda i: (p_off_blocks + i, 0, 0))
    if prev is None:
        return pl.pallas_call(
            _tc_add_first_body,
            grid=(n_blocks,),
            in_specs=[g_spec, pe_spec],
            out_specs=o_spec,
            out_shape=out_shape,
        )(g_part, pe)
    return pl.pallas_call(
        _tc_add_alias_body,
        grid=(n_blocks,),
        in_specs=[g_spec, pe_spec,
                  pl.BlockSpec(memory_space=pl.ANY)],
        out_specs=o_spec,
        out_shape=out_shape,
        input_output_aliases={2: 0},
    )(g_part, pe, prev)


def kernel(x, tok_table):
    batch, seq = x.shape
    assert seq == SEQ
    part_batch = batch // P_SPLIT
    part_rows = part_batch * SEQ
    assert part_rows % (NUM_WORKERS * CHUNK) == 0
    n_chunks = part_rows // (NUM_WORKERS * CHUNK)
    pe = _positional_encoding(SEQ, D_MODEL)
    sc_gather = _make_sc_gather(part_rows, n_chunks)
    x_flat = x.astype(jnp.int32).reshape(
        P_SPLIT, NUM_WORKERS, n_chunks, CHUNK)
    gathered = [sc_gather(x_flat[p], tok_table) for p in range(P_SPLIT)]
    out = None
    blocks_per_part = part_batch // TC_BK
    for p in range(P_SPLIT):
        out = _tc_add(batch, part_rows, p * blocks_per_part,
                      gathered[p], pe, out)
    return out


# speedup vs baseline: 1.2821x; 1.2821x over previous
import functools

import jax
import jax.numpy as jnp
from jax import lax
from jax.experimental import pallas as pl
from jax.experimental.pallas import tpu as pltpu
from jax.experimental.pallas import tpu_sc as plsc

D_MODEL = 128
SEQ = 50
LANES = 16
NUM_WORKERS = 32
BATCH_PER_CHUNK = 4
CHUNK = BATCH_PER_CHUNK * SEQ
NBUF = 4
P_SPLIT = 4


def _positional_encoding(seq, d_model):
    pos = jnp.arange(seq, dtype=jnp.float32)[:, None]
    i = jnp.arange(0, d_model, 2, dtype=jnp.float32)
    div = jnp.exp(-i * (jnp.log(10000.0) / d_model))
    ang = pos * div
    pe = jnp.zeros((seq, d_model), dtype=jnp.float32)
    pe = pe.at[:, 0::2].set(jnp.sin(ang))
    pe = pe.at[:, 1::2].set(jnp.cos(ang))
    return pe


def _make_sc_kernel(batch, n_chunks):
    mesh = plsc.VectorSubcoreMesh(core_axis_name="c", subcore_axis_name="s")
    n_dreg = D_MODEL // LANES
    assert n_chunks % NBUF == 0
    batch_per_w = batch // NUM_WORKERS

    @functools.partial(
        pl.kernel,
        mesh=mesh,
        out_type=jax.ShapeDtypeStruct((batch, SEQ, D_MODEL), jnp.float32),
        scratch_types=[
            pltpu.VMEM((SEQ, D_MODEL), jnp.float32),
        ]
        + [pltpu.VMEM((CHUNK,), jnp.int32) for _ in range(NBUF)]
        + [pltpu.VMEM((CHUNK, D_MODEL), jnp.float32) for _ in range(NBUF)]
        + [pltpu.SemaphoreType.DMA for _ in range(3 * NBUF)],
    )
    def sc_embed(x_hbm, tab_hbm, pe_hbm, out_hbm, pe_v, *bufs_sems):
        ibufs = bufs_sems[:NBUF]
        bufs = bufs_sems[NBUF:2 * NBUF]
        isem = bufs_sems[2 * NBUF:3 * NBUF]
        gsem = bufs_sems[3 * NBUF:4 * NBUF]
        ssem = bufs_sems[4 * NBUF:]
        cid = lax.axis_index("c")
        sid = lax.axis_index("s")
        w = sid * 2 + cid
        pltpu.sync_copy(pe_hbm, pe_v)
        batch_base = w * batch_per_w

        def start_idx(b, c):
            pltpu.async_copy(x_hbm.at[w, c], ibufs[b], isem[b])

        def wait_idx(b):
            pltpu.make_async_copy(x_hbm.at[w, 0], ibufs[b], isem[b]).wait()

        def start_gather(b):
            pltpu.async_copy(tab_hbm.at[ibufs[b]], bufs[b], gsem[b])

        def wait_gather(b):
            pltpu.make_async_copy(tab_hbm.at[ibufs[b]], bufs[b],
                                  gsem[b]).wait()

        def start_scatter(b, c):
            for k in range(BATCH_PER_CHUNK):
                pltpu.async_copy(
                    bufs[b].at[pl.ds(k * SEQ, SEQ)],
                    out_hbm.at[batch_base + c * BATCH_PER_CHUNK + k],
                    ssem[b])

        def wait_scatter(b):
            for _ in range(BATCH_PER_CHUNK):
                pltpu.make_async_copy(bufs[b].at[pl.ds(0, SEQ)],
                                      out_hbm.at[0], ssem[b]).wait()

        for b in range(NBUF):
            start_idx(b, b)
        for b in range(NBUF - 1):
            wait_idx(b)
            start_gather(b)

        def outer_body(g, carry):
            for b in range(NBUF):
                c = g * NBUF + b
                nb = (b + NBUF - 1) % NBUF
                @pl.when(c >= 1)
                def _():
                    wait_scatter(nb)

                @pl.when(c + NBUF - 1 < n_chunks)
                def _():
                    wait_idx(nb)
                    start_gather(nb)

                wait_gather(b)

                @pl.when(c + NBUF < n_chunks)
                def _():
                    start_idx(b, c + NBUF)

                def pe_body(s, carry2):
                    for j in range(BATCH_PER_CHUNK):
                        r = j * SEQ + s
                        for d in range(n_dreg):
                            sl = pl.ds(d * LANES, LANES)
                            plsc.addupdate(bufs[b].at[r, sl], pe_v[s, sl])
                    return carry2

                lax.fori_loop(0, SEQ, pe_body, 0)
                start_scatter(b, c)
            return carry

        lax.fori_loop(0, n_chunks // NBUF, outer_body, 0)
        wait_scatter((n_chunks - 1) % NBUF)

    return sc_embed


def kernel(x, tok_table):
    batch, seq = x.shape
    assert seq == SEQ
    part_batch = batch // P_SPLIT
    part_rows = part_batch * SEQ
    assert part_rows % (NUM_WORKERS * CHUNK) == 0
    n_chunks = part_rows // (NUM_WORKERS * CHUNK)
    x_flat = x.astype(jnp.int32).reshape(
        P_SPLIT, NUM_WORKERS, n_chunks, CHUNK)
    pe = _positional_encoding(SEQ, D_MODEL)
    sc_embed = _make_sc_kernel(part_batch, n_chunks)
    parts = [sc_embed(x_flat[p], tok_table, pe) for p in range(P_SPLIT)]
    return jnp.concatenate(parts, axis=0)
